# baseline (speedup 1.0000x reference)
import functools

import numpy as np

import jax
import jax.numpy as jnp
from jax import lax
from jax.experimental import pallas as pl
from jax.experimental.pallas import tpu as pltpu
from jax.experimental.pallas import tpu_sc as plsc

KNB = 32
FDIM = 128
QB = 512
CB = 128
NHALF = 4
BIGF = 3.0e38
BIGI = 2 ** 30


def _proj_kernel(x_ref, wa_ref, wc_ref, b_ref, a_ref, c_ref):
    x_q = x_ref[...]
    a_ref[...] = lax.dot_general(x_q, wa_ref[...],
                                 (((1,), (0,)), ((), ()))) + b_ref[...]
    c_ref[...] = lax.dot_general(x_q, wc_ref[...], (((1,), (0,)), ((), ())))


def _run_proj(np_total):
    nblocks = np_total // QB
    return pl.pallas_call(
        _proj_kernel,
        grid=(nblocks,),
        in_specs=[
            pl.BlockSpec((QB, FDIM), lambda i: (i, 0)),
            pl.BlockSpec((FDIM, FDIM), lambda i: (0, 0)),
            pl.BlockSpec((FDIM, FDIM), lambda i: (0, 0)),
            pl.BlockSpec((1, FDIM), lambda i: (0, 0)),
        ],
        out_specs=[
            pl.BlockSpec((QB, FDIM), lambda i: (i, 0)),
            pl.BlockSpec((QB, FDIM), lambda i: (i, 0)),
        ],
        out_shape=[
            jax.ShapeDtypeStruct((np_total, FDIM), jnp.float32),
            jax.ShapeDtypeStruct((np_total, FDIM), jnp.float32),
        ],
        compiler_params=pltpu.CompilerParams(
            dimension_semantics=("arbitrary",)),
    )


def _make_knn_kernel(qb_off):
    def knn_kernel(t0_ref, t1_ref, x_ref, bcol_ref, brow_ref, nbr_ref):
        qb = pl.program_id(0)
        q0 = (qb + qb_off) * QB
        x_q = x_ref[pl.ds(q0, QB), :]
        b_q = brow_ref[...]

        ones_row = jnp.ones((1, FDIM), jnp.float32)
        sq_q = lax.dot_general(ones_row, x_q * x_q,
                               (((1,), (1,)), ((), ())))

        qid = q0 + lax.broadcasted_iota(jnp.int32, (CB, QB), 1)
        cid_local = lax.broadcasted_iota(jnp.int32, (CB, QB), 0)

        def dist_tile(t):
            c0 = t * CB
            x_c = x_ref[pl.ds(c0, CB), :]
            b_c = bcol_ref[pl.ds(c0, CB), :]
            sq_c = jnp.sum(x_c * x_c, axis=1, keepdims=True)
            dotp = lax.dot_general(x_c, x_q,
                                   (((1,), (1,)), ((), ())))
            d2 = sq_c + sq_q - 2.0 * dotp
            cid = c0 + cid_local
            invalid = (b_c != b_q) | (cid == qid)
            return jnp.where(invalid, BIGF, d2), cid

        def tile_body(t, carry):
            best_d, best_i = carry
            d2, cid = dist_tile(t)
            comb_d = jnp.concatenate([best_d, d2], axis=0)
            comb_i = jnp.concatenate([best_i, cid], axis=0)
            nd, ni = [], []
            for _ in range(KNB):
                m = jnp.min(comb_d, axis=0, keepdims=True)
                sel = comb_d == m
                idk = jnp.min(jnp.where(sel, comb_i, BIGI), axis=0,
                              keepdims=True)
                nd.append(m)
                ni.append(idk)
                comb_d = jnp.where(sel, BIGF, comb_d)
            return (jnp.concatenate(nd, axis=0),
                    jnp.concatenate(ni, axis=0))

        init = (jnp.full((KNB, QB), BIGF, jnp.float32),
                jnp.zeros((KNB, QB), jnp.int32))
        _, best_i = lax.fori_loop(t0_ref[qb], t1_ref[qb], tile_body, init)
        nbr_ref[...] = best_i
    return knn_kernel


def _run_knn(np_total, np_half, qb_off):
    nblocks = np_half // QB
    grid_spec = pltpu.PrefetchScalarGridSpec(
        num_scalar_prefetch=2,
        grid=(nblocks,),
        in_specs=[
            pl.BlockSpec((np_total, FDIM), lambda qb, s0, s1: (0, 0)),
            pl.BlockSpec((np_total, 1), lambda qb, s0, s1: (0, 0)),
            pl.BlockSpec((1, QB), lambda qb, s0, s1: (0, qb + qb_off)),
        ],
        out_specs=pl.BlockSpec((KNB, QB), lambda qb, s0, s1: (0, qb)),
    )
    return pl.pallas_call(
        _make_knn_kernel(qb_off),
        grid_spec=grid_spec,
        out_shape=jax.ShapeDtypeStruct((KNB, np_half), jnp.int32),
        compiler_params=pltpu.CompilerParams(
            dimension_semantics=("arbitrary",)),
    )


def _make_edge_mean_sc(np_total, np_half):
    nc, ns, lanes = 2, 16, 16
    nw = nc * ns
    chunk = np_half // nw
    mesh = plsc.VectorSubcoreMesh(core_axis_name="c", subcore_axis_name="s",
                                  num_cores=nc, num_subcores=ns)

    gnodes = 4
    kbatch = 2
    pnodes = gnodes * kbatch
    nphase = chunk // pnodes
    nch = FDIM // lanes

    @functools.partial(
        pl.kernel,
        out_type=jax.ShapeDtypeStruct((np_half * FDIM,), jnp.float32),
        mesh=mesh,
        scratch_types=[
            pltpu.VMEM((chunk * KNB,), jnp.int32),
            pltpu.VMEM((chunk * FDIM,), jnp.float32),
            [pltpu.VMEM((gnodes * KNB, FDIM), jnp.float32)
             for _ in range(2 * kbatch)],
            [pltpu.VMEM((pnodes * FDIM,), jnp.float32) for _ in range(2)],
            pltpu.SemaphoreType.DMA,
            pltpu.SemaphoreType.DMA,
        ],
    )
    def edge_mean(a_hbm, c_hbm, nbr_hbm, out_hbm, idx_v, a_v, rows_bufs,
                  out_bufs, gsem, osem):
        wid = lax.axis_index("s") * nc + lax.axis_index("c")
        base = wid * chunk
        pltpu.sync_copy(nbr_hbm.at[pl.ds(base * KNB, chunk * KNB)], idx_v)
        pltpu.sync_copy(a_hbm.at[pl.ds(base * FDIM, chunk * FDIM)], a_v)

        def issue_phase(ph, par):
            for d in range(kbatch):
                g = ph * kbatch + d
                pltpu.async_copy(
                    c_hbm.at[idx_v.at[pl.ds(g * gnodes * KNB,
                                            gnodes * KNB)]],
                    rows_bufs[par * kbatch + d], gsem)

        def drain_phase():
            for d in range(kbatch):
                pltpu.make_async_copy(
                    c_hbm.at[pl.ds(0, gnodes * KNB)], rows_bufs[d],
                    gsem).wait()

        def compute_phase(ph, par):
            o_v = out_bufs[par]
            for d in range(kbatch):
                rows_v = rows_bufs[par * kbatch + d]

                def node_body(j, carry):
                    node = ph * pnodes + d * gnodes + j
                    a_off = node * FDIM
                    a_vecs = [a_v[pl.ds(a_off + ch * lanes, lanes)]
                              for ch in range(nch)]
                    accs = [jnp.zeros((lanes,), jnp.float32)
                            for _ in range(nch)]
                    for k in range(KNB):
                        for ch in range(nch):
                            v = rows_v[j * KNB + k,
                                       pl.ds(ch * lanes, lanes)]
                            accs[ch] = accs[ch] + jnp.maximum(
                                v + a_vecs[ch], 0.0)
                    o_off = (d * gnodes + j) * FDIM
                    for ch in range(nch):
                        o_v[pl.ds(o_off + ch * lanes, lanes)] = (
                            accs[ch] * (1.0 / KNB))
                    return carry
                lax.fori_loop(0, gnodes, node_body, 0)
            pltpu.async_copy(
                o_v, out_hbm.at[pl.ds((base + ph * pnodes) * FDIM,
                                      pnodes * FDIM)], osem)

        issue_phase(0, 0)

        def pair_body(ss, carry):
            ph0 = ss * 2
            drain_phase()
            issue_phase(ph0 + 1, 1)

            @pl.when(ss > 0)
            def _drain_outs():
                for _ in range(2):
                    pltpu.make_async_copy(
                        out_bufs[0],
                        out_hbm.at[pl.ds(0, pnodes * FDIM)], osem).wait()

            compute_phase(ph0, 0)
            drain_phase()

            @pl.when(ph0 + 2 < nphase)
            def _issue_next():
                issue_phase(ph0 + 2, 0)

            compute_phase(ph0 + 1, 1)
            return carry
        lax.fori_loop(0, nphase // 2, pair_body, 0)
        for _ in range(2):
            pltpu.make_async_copy(
                out_bufs[0], out_hbm.at[pl.ds(0, pnodes * FDIM)],
                osem).wait()

    return edge_mean


def kernel(x, batch, W, b):
    n, f = x.shape
    assert f == FDIM
    np_total = 10240
    assert np_total % (NHALF * QB) == 0 and np_total >= n
    pad = np_total - n
    np_half = np_total // NHALF

    xp = jnp.pad(x, ((0, pad), (0, 0)))
    bp = jnp.pad(batch, (0, pad), constant_values=jnp.int32(1 << 20))

    seg_start = jnp.searchsorted(bp, bp[::QB], side="left")
    seg_end = jnp.searchsorted(bp, bp[QB - 1::QB], side="right")
    t0_arr = (seg_start // CB).astype(jnp.int32)
    t1_arr = ((seg_end + CB - 1) // CB).astype(jnp.int32)

    bcol = bp.reshape(np_total, 1)
    brow = bp.reshape(1, np_total)
    bias2d = b.reshape(1, FDIM)
    wa = W[:FDIM] - W[FDIM:]
    wc = W[FDIM:]

    a_mat, c_mat = _run_proj(np_total)(xp, wa, wc, bias2d)

    splits = (2560, 2560, 2560, 2048, 512)
    assert sum(splits) == np_total
    outs = []
    off = 0
    for np_part in splits:
        nb0 = off // QB
        nbp = np_part // QB
        nbr_t = _run_knn(np_total, np_part, nb0)(
            t0_arr[nb0:nb0 + nbp], t1_arr[nb0:nb0 + nbp],
            xp, bcol, brow)
        nbr_flat = nbr_t.T.reshape(-1)
        a_flat = lax.dynamic_slice_in_dim(
            a_mat.reshape(-1), off * FDIM, np_part * FDIM)
        outs.append(_make_edge_mean_sc(np_total, np_part)(
            a_flat, c_mat, nbr_flat))
        off += np_part
    out = jnp.concatenate(outs).reshape(np_total, FDIM)
    return out[:n]


# speedup vs baseline: 1.2473x; 1.2473x over previous
import functools

import numpy as np

import jax
import jax.numpy as jnp
from jax import lax
from jax.experimental import pallas as pl
from jax.experimental.pallas import tpu as pltpu
from jax.experimental.pallas import tpu_sc as plsc

KNB = 32
FDIM = 128
QB = 512
CB = 128
NHALF = 4
BIGF = 3.0e38
BIGI = 2 ** 30


def _proj_kernel(x_ref, wa_ref, wc_ref, b_ref, a_ref, c_ref):
    x_q = x_ref[...]
    a_ref[...] = lax.dot_general(x_q, wa_ref[...],
                                 (((1,), (0,)), ((), ()))) + b_ref[...]
    c_ref[...] = lax.dot_general(x_q, wc_ref[...], (((1,), (0,)), ((), ())))


def _run_proj(np_total):
    nblocks = np_total // QB
    return pl.pallas_call(
        _proj_kernel,
        grid=(nblocks,),
        in_specs=[
            pl.BlockSpec((QB, FDIM), lambda i: (i, 0)),
            pl.BlockSpec((FDIM, FDIM), lambda i: (0, 0)),
            pl.BlockSpec((FDIM, FDIM), lambda i: (0, 0)),
            pl.BlockSpec((1, FDIM), lambda i: (0, 0)),
        ],
        out_specs=[
            pl.BlockSpec((QB, FDIM), lambda i: (i, 0)),
            pl.BlockSpec((QB, FDIM), lambda i: (i, 0)),
        ],
        out_shape=[
            jax.ShapeDtypeStruct((np_total, FDIM), jnp.float32),
            jax.ShapeDtypeStruct((np_total, FDIM), jnp.float32),
        ],
        compiler_params=pltpu.CompilerParams(
            dimension_semantics=("arbitrary",)),
    )


def _make_knn_kernel(qb_off):
    def knn_kernel(t0_ref, t1_ref, x_ref, bcol_ref, brow_ref, nbr_ref):
        qb = pl.program_id(0)
        q0 = (qb + qb_off) * QB
        x_q = x_ref[pl.ds(q0, QB), :]
        b_q = brow_ref[...]

        ones_row = jnp.ones((1, FDIM), jnp.float32)
        sq_q = lax.dot_general(ones_row, x_q * x_q,
                               (((1,), (1,)), ((), ())))

        qid = q0 + lax.broadcasted_iota(jnp.int32, (CB, QB), 1)
        cid_local = lax.broadcasted_iota(jnp.int32, (CB, QB), 0)

        def dist_tile(t):
            c0 = t * CB
            x_c = x_ref[pl.ds(c0, CB), :]
            b_c = bcol_ref[pl.ds(c0, CB), :]
            sq_c = jnp.sum(x_c * x_c, axis=1, keepdims=True)
            dotp = lax.dot_general(x_c, x_q,
                                   (((1,), (1,)), ((), ())))
            d2 = sq_c + sq_q - 2.0 * dotp
            cid = c0 + cid_local
            invalid = (b_c != b_q) | (cid == qid)
            return jnp.where(invalid, BIGF, d2), cid

        def tile_body(t, carry):
            best_d, best_i = carry
            d2, cid = dist_tile(t)
            comb_d = jnp.concatenate([best_d, d2], axis=0)
            comb_i = jnp.concatenate([best_i, cid], axis=0)
            nd, ni = [], []
            for _ in range(KNB):
                m = jnp.min(comb_d, axis=0, keepdims=True)
                sel = comb_d == m
                idk = jnp.min(jnp.where(sel, comb_i, BIGI), axis=0,
                              keepdims=True)
                nd.append(m)
                ni.append(idk)
                comb_d = jnp.where(sel, BIGF, comb_d)
            return (jnp.concatenate(nd, axis=0),
                    jnp.concatenate(ni, axis=0))

        init = (jnp.full((KNB, QB), BIGF, jnp.float32),
                jnp.zeros((KNB, QB), jnp.int32))
        _, best_i = lax.fori_loop(t0_ref[qb], t1_ref[qb], tile_body, init)
        nbr_ref[...] = best_i
    return knn_kernel


def _run_knn(np_total, np_half, qb_off):
    nblocks = np_half // QB
    grid_spec = pltpu.PrefetchScalarGridSpec(
        num_scalar_prefetch=2,
        grid=(nblocks,),
        in_specs=[
            pl.BlockSpec((np_total, FDIM), lambda qb, s0, s1: (0, 0)),
            pl.BlockSpec((np_total, 1), lambda qb, s0, s1: (0, 0)),
            pl.BlockSpec((1, QB), lambda qb, s0, s1: (0, qb + qb_off)),
        ],
        out_specs=pl.BlockSpec((KNB, QB), lambda qb, s0, s1: (0, qb)),
    )
    return pl.pallas_call(
        _make_knn_kernel(qb_off),
        grid_spec=grid_spec,
        out_shape=jax.ShapeDtypeStruct((KNB, np_half), jnp.int32),
        compiler_params=pltpu.CompilerParams(
            dimension_semantics=("arbitrary",)),
    )


def _make_edge_mean_sc(np_total, np_half):
    nc, ns, lanes = 2, 16, 16
    nw = nc * ns
    chunk = np_half // nw
    mesh = plsc.VectorSubcoreMesh(core_axis_name="c", subcore_axis_name="s",
                                  num_cores=nc, num_subcores=ns)

    gnodes = 4
    kbatch = 2
    pnodes = gnodes * kbatch
    nphase = chunk // pnodes
    nch = FDIM // lanes

    @functools.partial(
        pl.kernel,
        out_type=jax.ShapeDtypeStruct((np_half * FDIM,), jnp.float32),
        mesh=mesh,
        scratch_types=[
            pltpu.VMEM((chunk * KNB,), jnp.int32),
            pltpu.VMEM((chunk * FDIM,), jnp.float32),
            [pltpu.VMEM((gnodes * KNB, FDIM), jnp.float32)
             for _ in range(2 * kbatch)],
            [pltpu.VMEM((pnodes * FDIM,), jnp.float32) for _ in range(2)],
            pltpu.SemaphoreType.DMA,
            pltpu.SemaphoreType.DMA,
        ],
    )
    def edge_mean(a_hbm, c_hbm, nbr_hbm, out_hbm, idx_v, a_v, rows_bufs,
                  out_bufs, gsem, osem):
        wid = lax.axis_index("s") * nc + lax.axis_index("c")
        base = wid * chunk
        pltpu.sync_copy(nbr_hbm.at[pl.ds(base * KNB, chunk * KNB)], idx_v)
        pltpu.sync_copy(a_hbm.at[pl.ds(base * FDIM, chunk * FDIM)], a_v)

        def issue_phase(ph, par):
            for d in range(kbatch):
                g = ph * kbatch + d
                pltpu.async_copy(
                    c_hbm.at[idx_v.at[pl.ds(g * gnodes * KNB,
                                            gnodes * KNB)]],
                    rows_bufs[par * kbatch + d], gsem)

        def drain_phase():
            for d in range(kbatch):
                pltpu.make_async_copy(
                    c_hbm.at[pl.ds(0, gnodes * KNB)], rows_bufs[d],
                    gsem).wait()

        def compute_phase(ph, par):
            o_v = out_bufs[par]
            for d in range(kbatch):
                rows_v = rows_bufs[par * kbatch + d]

                def node_body(j, carry):
                    node = ph * pnodes + d * gnodes + j
                    a_off = node * FDIM
                    a_vecs = [a_v[pl.ds(a_off + ch * lanes, lanes)]
                              for ch in range(nch)]
                    accs = [jnp.zeros((lanes,), jnp.float32)
                            for _ in range(nch)]
                    for k in range(KNB):
                        for ch in range(nch):
                            v = rows_v[j * KNB + k,
                                       pl.ds(ch * lanes, lanes)]
                            accs[ch] = accs[ch] + jnp.maximum(
                                v + a_vecs[ch], 0.0)
                    o_off = (d * gnodes + j) * FDIM
                    for ch in range(nch):
                        o_v[pl.ds(o_off + ch * lanes, lanes)] = (
                            accs[ch] * (1.0 / KNB))
                    return carry
                lax.fori_loop(0, gnodes, node_body, 0)
            pltpu.async_copy(
                o_v, out_hbm.at[pl.ds((base + ph * pnodes) * FDIM,
                                      pnodes * FDIM)], osem)

        issue_phase(0, 0)

        def pair_body(ss, carry):
            ph0 = ss * 2
            drain_phase()
            issue_phase(ph0 + 1, 1)

            @pl.when(ss > 0)
            def _drain_outs():
                for _ in range(2):
                    pltpu.make_async_copy(
                        out_bufs[0],
                        out_hbm.at[pl.ds(0, pnodes * FDIM)], osem).wait()

            compute_phase(ph0, 0)
            drain_phase()

            @pl.when(ph0 + 2 < nphase)
            def _issue_next():
                issue_phase(ph0 + 2, 0)

            compute_phase(ph0 + 1, 1)
            return carry
        lax.fori_loop(0, nphase // 2, pair_body, 0)
        for _ in range(2):
            pltpu.make_async_copy(
                out_bufs[0], out_hbm.at[pl.ds(0, pnodes * FDIM)],
                osem).wait()

    return edge_mean


def kernel(x, batch, W, b):
    n, f = x.shape
    assert f == FDIM
    np_total = 10240
    assert np_total % (NHALF * QB) == 0 and np_total >= n
    pad = np_total - n
    np_half = np_total // NHALF

    xp = jnp.pad(x, ((0, pad), (0, 0)))
    bp = jnp.pad(batch, (0, pad), constant_values=jnp.int32(1 << 20))

    seg_start = jnp.searchsorted(bp, bp[::QB], side="left")
    seg_end = jnp.searchsorted(bp, bp[QB - 1::QB], side="right")
    t0_arr = (seg_start // CB).astype(jnp.int32)
    t1_arr = ((seg_end + CB - 1) // CB).astype(jnp.int32)

    bcol = bp.reshape(np_total, 1)
    brow = bp.reshape(1, np_total)
    bias2d = b.reshape(1, FDIM)
    wa = W[:FDIM] - W[FDIM:]
    wc = W[FDIM:]

    a_mat, c_mat = _run_proj(np_total)(xp, wa, wc, bias2d)

    splits = (3072, 3072, 3072, 1024)
    assert sum(splits) == np_total
    outs = []
    off = 0
    for np_part in splits:
        nb0 = off // QB
        nbp = np_part // QB
        nbr_t = _run_knn(np_total, np_part, nb0)(
            t0_arr[nb0:nb0 + nbp], t1_arr[nb0:nb0 + nbp],
            xp, bcol, brow)
        nbr_flat = nbr_t.T.reshape(-1)
        a_flat = lax.dynamic_slice_in_dim(
            a_mat.reshape(-1), off * FDIM, np_part * FDIM)
        outs.append(_make_edge_mean_sc(np_total, np_part)(
            a_flat, c_mat, nbr_flat))
        off += np_part
    out = jnp.concatenate(outs).reshape(np_total, FDIM)
    return out[:n]
